# baseline (speedup 1.0000x reference)
import functools

import jax
import jax.numpy as jnp
from jax import lax
from jax.experimental.compute_on import compute_on
from jax.experimental import pallas as pl
from jax.experimental.pallas import tpu as pltpu
from jax.experimental.pallas import tpu_sc as plsc

N = 10000
E = 320000
D = 128
NC = 2
NS = 16
NW = NC * NS
CHUNK = 128
NCHUNK = E // (NW * CHUNK)
EPW = NCHUNK * CHUNK
NEXTRA = (E - NW * EPW) // CHUNK
RPT = N // NS
MBLK = 1000

_MESH = plsc.VectorSubcoreMesh(
    core_axis_name="c", subcore_axis_name="s", num_cores=NC, num_subcores=NS
)
_PREC = jax.lax.Precision.HIGHEST


COVER = RPT + 7


def _tile_row_start(sid):
    return pl.multiple_of((sid * RPT) // 8 * 8, 8)


def _start_init_acc(rows, acc, sid, sem):
    def zrow(i, carry):
        for j in range(D // 16):
            rows[i, pl.ds(j * 16, 16)] = jnp.zeros((16,), jnp.float32)
        return carry
    lax.fori_loop(0, CHUNK, zrow, 0)

    base = _tile_row_start(sid)
    nfull, tail = COVER // CHUNK, COVER % CHUNK
    for t in range(nfull):
        pltpu.async_copy(rows, acc.at[pl.ds(base + t * CHUNK, CHUNK)], sem)
    if tail:
        pltpu.async_copy(rows.at[pl.ds(0, tail)],
                         acc.at[pl.ds(base + nfull * CHUNK, tail)], sem)


def _wait_init_acc(rows, acc, sid, sem):
    base = _tile_row_start(sid)
    nfull, tail = COVER // CHUNK, COVER % CHUNK
    for t in range(nfull):
        pltpu.make_async_copy(
            rows, acc.at[pl.ds(base + t * CHUNK, CHUNK)], sem).wait()
    if tail:
        pltpu.make_async_copy(
            rows.at[pl.ds(0, tail)],
            acc.at[pl.ds(base + nfull * CHUNK, tail)], sem).wait()


@functools.partial(
    pl.kernel,
    out_type=jax.ShapeDtypeStruct((NC, N, D), jnp.float32),
    mesh=_MESH,
    scratch_types=(
        [pltpu.VMEM((CHUNK,), jnp.int32)] * 8
        + [
            pltpu.VMEM((CHUNK, D), jnp.float32),
            pltpu.VMEM((CHUNK, D), jnp.float32),
            pltpu.VMEM_SHARED((N, D), jnp.float32),
        ]
        + [pltpu.SemaphoreType.DMA] * 8
    ),
)
def _agg(h_hbm, src_hbm, dst_hbm, out_hbm,
         sv0, sv1, sv2, sv3, dv0, dv1, dv2, dv3, rows0, rows1, acc,
         gsem0, gsem1, ssem0, ssem1, isem0, isem1, isem2, isem3):
    cid = lax.axis_index("c")
    sid = lax.axis_index("s")
    wid = sid * NC + cid
    srcv = (sv0, sv1, sv2, sv3)
    dstv = (dv0, dv1, dv2, dv3)
    rows = (rows0, rows1)
    gsem = (gsem0, gsem1)
    ssem = (ssem0, ssem1)
    isem = (isem0, isem1, isem2, isem3)

    ebase = wid * EPW
    last = NCHUNK - 1

    def idx_off(c):
        return pl.ds(ebase + c * CHUNK, CHUNK)

    def prefetch_idx(c, q):
        cc = jnp.minimum(c, last)
        pltpu.async_copy(src_hbm.at[idx_off(cc)], srcv[q], isem[q])
        pltpu.async_copy(dst_hbm.at[idx_off(cc)], dstv[q], isem[q])

    def wait_idx(c, q):
        cc = jnp.minimum(c, last)
        pltpu.make_async_copy(src_hbm.at[idx_off(cc)], srcv[q], isem[q]).wait()
        pltpu.make_async_copy(dst_hbm.at[idx_off(cc)], dstv[q], isem[q]).wait()

    def start_gather(c, b, q):
        del c
        pltpu.async_copy(h_hbm.at[srcv[q]], rows[b], gsem[b])

    def wait_gather(c, b, q):
        del c
        pltpu.make_async_copy(h_hbm.at[srcv[q]], rows[b], gsem[b]).wait()

    def start_scatter(c, b, q):
        del c
        pltpu.async_copy(rows[b], acc.at[dstv[q]], ssem[b], add=True)

    def wait_scatter(c, b, q):
        del c
        pltpu.make_async_copy(rows[b], acc.at[dstv[q]], ssem[b]).wait()

    for q in range(4):
        prefetch_idx(q, q)
    _start_init_acc(rows0, acc, sid, ssem0)
    _wait_init_acc(rows0, acc, sid, ssem0)
    wait_idx(0, 0)
    start_gather(0, 0, 0)
    wait_idx(1, 1)
    start_gather(1, 1, 1)
    plsc.subcore_barrier()
    wait_gather(0, 0, 0)
    start_scatter(0, 0, 0)

    def phase(c, b, q):
        q1 = (q - 1) & 3
        q2 = (q + 2) & 3
        wait_scatter(c - 2, b, q2)
        wait_idx(c, q)
        start_gather(c, b, q)
        prefetch_idx(c + 2, q2)
        wait_gather(c - 1, 1 - b, q1)
        start_scatter(c - 1, 1 - b, q1)

    def body(k, carry):
        c = 4 * k + 2
        phase(c, 0, 2)
        phase(c + 1, 1, 3)
        phase(c + 2, 0, 0)
        phase(c + 3, 1, 1)
        return carry

    lax.fori_loop(0, (NCHUNK - 2) // 4, body, 0)

    wait_gather(last, 1, last & 3)
    start_scatter(last, 1, last & 3)
    wait_scatter(last - 1, 0, (last - 1) & 3)
    wait_idx(last, (last + 1) & 3)
    wait_idx(last, (last + 2) & 3)

    @pl.when(wid < NEXTRA)
    def _extra():
        xoff = pl.ds(NW * EPW + wid * CHUNK, CHUNK)
        pltpu.sync_copy(src_hbm.at[xoff], sv0)
        pltpu.sync_copy(dst_hbm.at[xoff], dv0)
        pltpu.async_copy(h_hbm.at[sv0], rows0, gsem0)
        pltpu.make_async_copy(h_hbm.at[sv0], rows0, gsem0).wait()
        pltpu.async_copy(rows0, acc.at[dv0], ssem0, add=True)
        pltpu.make_async_copy(rows0, acc.at[dv0], ssem0).wait()

    wait_scatter(last, 1, last & 3)
    plsc.subcore_barrier()
    base = _tile_row_start(sid)
    pltpu.sync_copy(acc.at[pl.ds(base, COVER)],
                    out_hbm.at[cid, pl.ds(base, COVER)])


@functools.partial(
    pl.kernel,
    out_type=jax.ShapeDtypeStruct((NC, N, D), jnp.float32),
    mesh=_MESH,
    scratch_types=[
        pltpu.VMEM((NCHUNK, 1, CHUNK), jnp.int32),
        pltpu.VMEM((1, CHUNK), jnp.int32),
        pltpu.VMEM((CHUNK, D), jnp.float32),
        pltpu.VMEM_SHARED((N, D), jnp.float32),
        pltpu.SemaphoreType.DMA,
        pltpu.SemaphoreType.DMA,
    ],
)
def _sc_degree(dst_hbm, out_hbm, dstv, dstx, rows, acc, ssem0, ssem1):
    cid = lax.axis_index("c")
    sid = lax.axis_index("s")
    wid = sid * NC + cid
    ssem = (ssem0, ssem1)

    pltpu.async_copy(dst_hbm.at[pl.ds(wid * NCHUNK, NCHUNK)], dstv, ssem1)
    _start_init_acc(rows, acc, sid, ssem0)
    _wait_init_acc(rows, acc, sid, ssem0)

    def orow(i, carry):
        rows[i, pl.ds(0, 16)] = jnp.ones((16,), jnp.float32)
        return carry
    lax.fori_loop(0, CHUNK, orow, 0)

    pltpu.make_async_copy(
        dst_hbm.at[pl.ds(wid * NCHUNK, NCHUNK)], dstv, ssem1).wait()
    plsc.subcore_barrier()

    def start_scatter(c, b):
        pltpu.async_copy(rows, acc.at[dstv.at[c, 0]], ssem[b], add=True)

    def wait_scatter(c, b):
        pltpu.make_async_copy(rows, acc.at[dstv.at[c, 0]], ssem[b]).wait()

    start_scatter(0, 0)
    start_scatter(1, 1)

    def phase(c, b):
        wait_scatter(c - 2, b)
        start_scatter(c, b)

    def body(k, carry):
        phase(2 * k + 2, 0)
        phase(2 * k + 3, 1)
        return carry

    lax.fori_loop(0, (NCHUNK - 2) // 2, body, 0)
    wait_scatter(NCHUNK - 2, 0)

    @pl.when(wid < NEXTRA)
    def _extra():
        pltpu.sync_copy(dst_hbm.at[NW * NCHUNK + wid], dstx)
        pltpu.async_copy(rows, acc.at[dstx.at[0]], ssem0, add=True)
        pltpu.make_async_copy(rows, acc.at[dstx.at[0]], ssem0).wait()

    wait_scatter(NCHUNK - 1, 1)
    plsc.subcore_barrier()
    base = _tile_row_start(sid)
    pltpu.sync_copy(acc.at[pl.ds(base, COVER)],
                    out_hbm.at[cid, pl.ds(base, COVER)])


_DINV_SPEC = pl.BlockSpec((MBLK, 1), lambda i: (i, 0))


def _dv(dinv_ref):
    return dinv_ref[...]


def _dinv_body(degp_ref, o_ref):
    deg = degp_ref[0, :, 0:1] + degp_ref[1, :, 0:1] + 1.0
    o_ref[...] = lax.rsqrt(jnp.maximum(deg, 1.0))


_dinv_call = pl.pallas_call(
    _dinv_body,
    grid=(N // MBLK,),
    in_specs=[pl.BlockSpec((NC, MBLK, D), lambda i: (0, i, 0))],
    out_specs=_DINV_SPEC,
    out_shape=jax.ShapeDtypeStruct((N, 1), jnp.float32),
)


def _pre_mm_body(x_ref, dinv_ref, w_ref, o_ref):
    o_ref[...] = jnp.dot(x_ref[...] * _dv(dinv_ref), w_ref[...],
                         preferred_element_type=jnp.float32, precision=_PREC)


_pre_mm_call = pl.pallas_call(
    _pre_mm_body,
    grid=(N // MBLK,),
    in_specs=[
        pl.BlockSpec((MBLK, D), lambda i: (i, 0)),
        _DINV_SPEC,
        pl.BlockSpec((D, D), lambda i: (0, 0)),
    ],
    out_specs=pl.BlockSpec((MBLK, D), lambda i: (i, 0)),
    out_shape=jax.ShapeDtypeStruct((N, D), jnp.float32),
)


def _comb_mm_body(p_ref, h_ref, dinv_ref, b_ref, w_ref, o_ref):
    dv = _dv(dinv_ref)
    t = (p_ref[0] + p_ref[1] + h_ref[...]) * dv + b_ref[...]
    t = jnp.maximum(t, 0.0)
    o_ref[...] = jnp.dot(t * dv, w_ref[...],
                         preferred_element_type=jnp.float32, precision=_PREC)


_comb_mm_call = pl.pallas_call(
    _comb_mm_body,
    grid=(N // MBLK,),
    in_specs=[
        pl.BlockSpec((NC, MBLK, D), lambda i: (0, i, 0)),
        pl.BlockSpec((MBLK, D), lambda i: (i, 0)),
        _DINV_SPEC,
        pl.BlockSpec((1, D), lambda i: (0, 0)),
        pl.BlockSpec((D, D), lambda i: (0, 0)),
    ],
    out_specs=pl.BlockSpec((MBLK, D), lambda i: (i, 0)),
    out_shape=jax.ShapeDtypeStruct((N, D), jnp.float32),
)


def _comb_mm2_body(p_ref, h_ref, dinv_ref, b_ref, w_ref, o3_ref, o5_ref):
    dv = _dv(dinv_ref)
    t = (p_ref[0, :, 0:64] + p_ref[1, :, 0:64] + h_ref[:, 0:64]) * dv \
        + b_ref[...]
    t = jnp.maximum(t, 0.0)
    r = jnp.dot(t * dv, w_ref[...],
                preferred_element_type=jnp.float32, precision=_PREC)
    o3_ref[...] = r[:, 0:D]
    o5_ref[...] = r[:, D:2 * D]


_comb_mm2_call = pl.pallas_call(
    _comb_mm2_body,
    grid=(N // MBLK,),
    in_specs=[
        pl.BlockSpec((NC, MBLK, D), lambda i: (0, i, 0)),
        pl.BlockSpec((MBLK, D), lambda i: (i, 0)),
        _DINV_SPEC,
        pl.BlockSpec((1, 64), lambda i: (0, 0)),
        pl.BlockSpec((64, 2 * D), lambda i: (0, 0)),
    ],
    out_specs=(
        pl.BlockSpec((MBLK, D), lambda i: (i, 0)),
        pl.BlockSpec((MBLK, D), lambda i: (i, 0)),
    ),
    out_shape=(
        jax.ShapeDtypeStruct((N, D), jnp.float32),
        jax.ShapeDtypeStruct((N, D), jnp.float32),
    ),
)


def _s_body(p_ref, h_ref, dinv_ref, b_ref, o_ref):
    dv = _dv(dinv_ref)
    t = (p_ref[0, :, 0:64] + p_ref[1, :, 0:64] + h_ref[:, 0:64]) * dv \
        + b_ref[...]
    o_ref[...] = jnp.maximum(t, 0.0)


_s_call = pl.pallas_call(
    _s_body,
    grid=(N // MBLK,),
    in_specs=[
        pl.BlockSpec((NC, MBLK, D), lambda i: (0, i, 0)),
        pl.BlockSpec((MBLK, D), lambda i: (i, 0)),
        _DINV_SPEC,
        pl.BlockSpec((1, 64), lambda i: (0, 0)),
    ],
    out_specs=pl.BlockSpec((MBLK, 64), lambda i: (i, 0)),
    out_shape=jax.ShapeDtypeStruct((N, 64), jnp.float32),
)


def _xhat_body(p_ref, h_ref, dinv_ref, b_ref, o_ref):
    dv = _dv(dinv_ref)
    o_ref[...] = (p_ref[0] + p_ref[1] + h_ref[...]) * dv + b_ref[...]


_xhat_call = pl.pallas_call(
    _xhat_body,
    grid=(N // MBLK,),
    in_specs=[
        pl.BlockSpec((NC, MBLK, D), lambda i: (0, i, 0)),
        pl.BlockSpec((MBLK, D), lambda i: (i, 0)),
        _DINV_SPEC,
        pl.BlockSpec((1, D), lambda i: (0, 0)),
    ],
    out_specs=pl.BlockSpec((MBLK, D), lambda i: (i, 0)),
    out_shape=jax.ShapeDtypeStruct((N, D), jnp.float32),
)


def _ahat_body(si_ref, sj_ref, o_ref):
    o_ref[...] = lax.dot_general(
        si_ref[...], sj_ref[...], (((1,), (1,)), ((), ())),
        preferred_element_type=jnp.float32, precision=_PREC)


ABLK = 1024

_ahat_call = pl.pallas_call(
    _ahat_body,
    grid=(pl.cdiv(N, ABLK), pl.cdiv(N, ABLK)),
    in_specs=[
        pl.BlockSpec((ABLK, 64), lambda i, j: (i, 0)),
        pl.BlockSpec((ABLK, 64), lambda i, j: (j, 0)),
    ],
    out_specs=pl.BlockSpec((ABLK, ABLK), lambda i, j: (i, j)),
    out_shape=jax.ShapeDtypeStruct((N, N), jnp.float32),
)


def _agg_async(h, src, dst):
    with compute_on('tpu_sparsecore'):
        return _agg(h, src, dst)


def kernel(x, edge_index, W1, b1, W2, b2, W3, b3, W4, b4, W5, b5):
    src = edge_index[0]
    dst = edge_index[1]

    with compute_on('tpu_sparsecore'):
        degp = _sc_degree(dst.reshape(E // CHUNK, 1, CHUNK))
    dinv = _dinv_call(degp)

    z64 = jnp.zeros((64, 64), jnp.float32)
    W2p = jnp.concatenate([W2, jnp.zeros((D, 64), jnp.float32)], axis=1)
    Wcat = jnp.concatenate([W3, W5, z64], axis=1)

    h1 = _pre_mm_call(x, dinv, W1)
    p1 = _agg_async(h1, src, dst)
    h2 = _comb_mm_call(p1, h1, dinv, b1.reshape(1, -1), W2p)
    p2 = _agg_async(h2, src, dst)

    h3, h5 = _comb_mm2_call(p2, h2, dinv, b2.reshape(1, -1), Wcat)
    p5 = _agg_async(h5, src, dst)
    s = _s_call(p5, h5, dinv, b5.reshape(1, -1))

    p3 = _agg_async(h3, src, dst)
    a_hat = _ahat_call(s, s)
    h4 = _comb_mm_call(p3, h3, dinv, b3.reshape(1, -1), W4)
    p4 = _agg_async(h4, src, dst)
    x_hat = _xhat_call(p4, h4, dinv, b4.reshape(1, -1))
    return (x_hat, a_hat)


# speedup vs baseline: 1.0009x; 1.0009x over previous
import functools

import jax
import jax.numpy as jnp
from jax import lax
from jax.experimental import pallas as pl
from jax.experimental.pallas import tpu as pltpu
from jax.experimental.pallas import tpu_sc as plsc

N = 10000
E = 320000
D = 128
NC = 2
NS = 16
NW = NC * NS
CHUNK = 128
NCHUNK = E // (NW * CHUNK)
EPW = NCHUNK * CHUNK
NEXTRA = (E - NW * EPW) // CHUNK
RPT = N // NS
MBLK = 1000

_MESH = plsc.VectorSubcoreMesh(
    core_axis_name="c", subcore_axis_name="s", num_cores=NC, num_subcores=NS
)
_PREC = jax.lax.Precision.HIGHEST


COVER = RPT + 7


def _tile_row_start(sid):
    return pl.multiple_of((sid * RPT) // 8 * 8, 8)


def _start_init_acc(rows, acc, sid, sem):
    def zrow(i, carry):
        for j in range(D // 16):
            rows[i, pl.ds(j * 16, 16)] = jnp.zeros((16,), jnp.float32)
        return carry
    lax.fori_loop(0, CHUNK, zrow, 0)

    base = _tile_row_start(sid)
    nfull, tail = COVER // CHUNK, COVER % CHUNK
    for t in range(nfull):
        pltpu.async_copy(rows, acc.at[pl.ds(base + t * CHUNK, CHUNK)], sem)
    if tail:
        pltpu.async_copy(rows.at[pl.ds(0, tail)],
                         acc.at[pl.ds(base + nfull * CHUNK, tail)], sem)


def _wait_init_acc(rows, acc, sid, sem):
    base = _tile_row_start(sid)
    nfull, tail = COVER // CHUNK, COVER % CHUNK
    for t in range(nfull):
        pltpu.make_async_copy(
            rows, acc.at[pl.ds(base + t * CHUNK, CHUNK)], sem).wait()
    if tail:
        pltpu.make_async_copy(
            rows.at[pl.ds(0, tail)],
            acc.at[pl.ds(base + nfull * CHUNK, tail)], sem).wait()


@functools.partial(
    pl.kernel,
    out_type=jax.ShapeDtypeStruct((NC, N, D), jnp.float32),
    mesh=_MESH,
    scratch_types=(
        [pltpu.VMEM((CHUNK,), jnp.int32)] * 8
        + [
            pltpu.VMEM((CHUNK, D), jnp.float32),
            pltpu.VMEM((CHUNK, D), jnp.float32),
            pltpu.VMEM_SHARED((N, D), jnp.float32),
        ]
        + [pltpu.SemaphoreType.DMA] * 8
    ),
)
def _agg(h_hbm, src_hbm, dst_hbm, out_hbm,
         sv0, sv1, sv2, sv3, dv0, dv1, dv2, dv3, rows0, rows1, acc,
         gsem0, gsem1, ssem0, ssem1, isem0, isem1, isem2, isem3):
    cid = lax.axis_index("c")
    sid = lax.axis_index("s")
    wid = sid * NC + cid
    srcv = (sv0, sv1, sv2, sv3)
    dstv = (dv0, dv1, dv2, dv3)
    rows = (rows0, rows1)
    gsem = (gsem0, gsem1)
    ssem = (ssem0, ssem1)
    isem = (isem0, isem1, isem2, isem3)

    ebase = wid * EPW
    last = NCHUNK - 1

    def idx_off(c):
        return pl.ds(ebase + c * CHUNK, CHUNK)

    def prefetch_idx(c, q):
        cc = jnp.minimum(c, last)
        pltpu.async_copy(src_hbm.at[idx_off(cc)], srcv[q], isem[q])
        pltpu.async_copy(dst_hbm.at[idx_off(cc)], dstv[q], isem[q])

    def wait_idx(c, q):
        cc = jnp.minimum(c, last)
        pltpu.make_async_copy(src_hbm.at[idx_off(cc)], srcv[q], isem[q]).wait()
        pltpu.make_async_copy(dst_hbm.at[idx_off(cc)], dstv[q], isem[q]).wait()

    def start_gather(c, b, q):
        del c
        pltpu.async_copy(h_hbm.at[srcv[q]], rows[b], gsem[b])

    def wait_gather(c, b, q):
        del c
        pltpu.make_async_copy(h_hbm.at[srcv[q]], rows[b], gsem[b]).wait()

    def start_scatter(c, b, q):
        del c
        pltpu.async_copy(rows[b], acc.at[dstv[q]], ssem[b], add=True)

    def wait_scatter(c, b, q):
        del c
        pltpu.make_async_copy(rows[b], acc.at[dstv[q]], ssem[b]).wait()

    for q in range(4):
        prefetch_idx(q, q)
    _start_init_acc(rows0, acc, sid, ssem0)
    _wait_init_acc(rows0, acc, sid, ssem0)
    wait_idx(0, 0)
    start_gather(0, 0, 0)
    wait_idx(1, 1)
    start_gather(1, 1, 1)
    plsc.subcore_barrier()
    wait_gather(0, 0, 0)
    start_scatter(0, 0, 0)

    def phase(c, b, q):
        q1 = (q - 1) & 3
        q2 = (q + 2) & 3
        wait_scatter(c - 2, b, q2)
        wait_idx(c, q)
        start_gather(c, b, q)
        prefetch_idx(c + 2, q2)
        wait_gather(c - 1, 1 - b, q1)
        start_scatter(c - 1, 1 - b, q1)

    def body(k, carry):
        c = 4 * k + 2
        phase(c, 0, 2)
        phase(c + 1, 1, 3)
        phase(c + 2, 0, 0)
        phase(c + 3, 1, 1)
        return carry

    lax.fori_loop(0, (NCHUNK - 2) // 4, body, 0)

    wait_gather(last, 1, last & 3)
    start_scatter(last, 1, last & 3)
    wait_scatter(last - 1, 0, (last - 1) & 3)
    wait_idx(last, (last + 1) & 3)
    wait_idx(last, (last + 2) & 3)

    @pl.when(wid < NEXTRA)
    def _extra():
        xoff = pl.ds(NW * EPW + wid * CHUNK, CHUNK)
        pltpu.sync_copy(src_hbm.at[xoff], sv0)
        pltpu.sync_copy(dst_hbm.at[xoff], dv0)
        pltpu.async_copy(h_hbm.at[sv0], rows0, gsem0)
        pltpu.make_async_copy(h_hbm.at[sv0], rows0, gsem0).wait()
        pltpu.async_copy(rows0, acc.at[dv0], ssem0, add=True)
        pltpu.make_async_copy(rows0, acc.at[dv0], ssem0).wait()

    wait_scatter(last, 1, last & 3)
    plsc.subcore_barrier()
    base = _tile_row_start(sid)
    pltpu.sync_copy(acc.at[pl.ds(base, COVER)],
                    out_hbm.at[cid, pl.ds(base, COVER)])


@functools.partial(
    pl.kernel,
    out_type=jax.ShapeDtypeStruct((NC, N, D), jnp.float32),
    mesh=_MESH,
    scratch_types=[
        pltpu.VMEM((NCHUNK, 1, CHUNK), jnp.int32),
        pltpu.VMEM((1, CHUNK), jnp.int32),
        pltpu.VMEM((CHUNK, D), jnp.float32),
        pltpu.VMEM_SHARED((N, D), jnp.float32),
        pltpu.SemaphoreType.DMA,
        pltpu.SemaphoreType.DMA,
    ],
)
def _sc_degree(dst_hbm, out_hbm, dstv, dstx, rows, acc, ssem0, ssem1):
    cid = lax.axis_index("c")
    sid = lax.axis_index("s")
    wid = sid * NC + cid
    ssem = (ssem0, ssem1)

    pltpu.async_copy(dst_hbm.at[pl.ds(wid * NCHUNK, NCHUNK)], dstv, ssem1)
    _start_init_acc(rows, acc, sid, ssem0)
    _wait_init_acc(rows, acc, sid, ssem0)

    def orow(i, carry):
        rows[i, pl.ds(0, 16)] = jnp.ones((16,), jnp.float32)
        return carry
    lax.fori_loop(0, CHUNK, orow, 0)

    pltpu.make_async_copy(
        dst_hbm.at[pl.ds(wid * NCHUNK, NCHUNK)], dstv, ssem1).wait()
    plsc.subcore_barrier()

    def start_scatter(c, b):
        pltpu.async_copy(rows, acc.at[dstv.at[c, 0]], ssem[b], add=True)

    def wait_scatter(c, b):
        pltpu.make_async_copy(rows, acc.at[dstv.at[c, 0]], ssem[b]).wait()

    start_scatter(0, 0)
    start_scatter(1, 1)

    def phase(c, b):
        wait_scatter(c - 2, b)
        start_scatter(c, b)

    def body(k, carry):
        phase(2 * k + 2, 0)
        phase(2 * k + 3, 1)
        return carry

    lax.fori_loop(0, (NCHUNK - 2) // 2, body, 0)
    wait_scatter(NCHUNK - 2, 0)

    @pl.when(wid < NEXTRA)
    def _extra():
        pltpu.sync_copy(dst_hbm.at[NW * NCHUNK + wid], dstx)
        pltpu.async_copy(rows, acc.at[dstx.at[0]], ssem0, add=True)
        pltpu.make_async_copy(rows, acc.at[dstx.at[0]], ssem0).wait()

    wait_scatter(NCHUNK - 1, 1)
    plsc.subcore_barrier()
    base = _tile_row_start(sid)
    pltpu.sync_copy(acc.at[pl.ds(base, COVER)],
                    out_hbm.at[cid, pl.ds(base, COVER)])


_DINV_SPEC = pl.BlockSpec((MBLK, 1), lambda i: (i, 0))


def _dv(dinv_ref):
    return dinv_ref[...]


def _dinv_body(degp_ref, o_ref):
    deg = degp_ref[0, :, 0:1] + degp_ref[1, :, 0:1] + 1.0
    o_ref[...] = lax.rsqrt(jnp.maximum(deg, 1.0))


_dinv_call = pl.pallas_call(
    _dinv_body,
    grid=(N // MBLK,),
    in_specs=[pl.BlockSpec((NC, MBLK, D), lambda i: (0, i, 0))],
    out_specs=_DINV_SPEC,
    out_shape=jax.ShapeDtypeStruct((N, 1), jnp.float32),
)


def _pre_mm_body(x_ref, dinv_ref, w_ref, o_ref):
    o_ref[...] = jnp.dot(x_ref[...] * _dv(dinv_ref), w_ref[...],
                         preferred_element_type=jnp.float32, precision=_PREC)


_pre_mm_call = pl.pallas_call(
    _pre_mm_body,
    grid=(N // MBLK,),
    in_specs=[
        pl.BlockSpec((MBLK, D), lambda i: (i, 0)),
        _DINV_SPEC,
        pl.BlockSpec((D, D), lambda i: (0, 0)),
    ],
    out_specs=pl.BlockSpec((MBLK, D), lambda i: (i, 0)),
    out_shape=jax.ShapeDtypeStruct((N, D), jnp.float32),
)


def _comb_mm_body(p_ref, h_ref, dinv_ref, b_ref, w_ref, o_ref):
    dv = _dv(dinv_ref)
    t = (p_ref[0] + p_ref[1] + h_ref[...]) * dv + b_ref[...]
    t = jnp.maximum(t, 0.0)
    o_ref[...] = jnp.dot(t * dv, w_ref[...],
                         preferred_element_type=jnp.float32, precision=_PREC)


_comb_mm_call = pl.pallas_call(
    _comb_mm_body,
    grid=(N // MBLK,),
    in_specs=[
        pl.BlockSpec((NC, MBLK, D), lambda i: (0, i, 0)),
        pl.BlockSpec((MBLK, D), lambda i: (i, 0)),
        _DINV_SPEC,
        pl.BlockSpec((1, D), lambda i: (0, 0)),
        pl.BlockSpec((D, D), lambda i: (0, 0)),
    ],
    out_specs=pl.BlockSpec((MBLK, D), lambda i: (i, 0)),
    out_shape=jax.ShapeDtypeStruct((N, D), jnp.float32),
)


def _comb_mm2_body(p_ref, h_ref, dinv_ref, b_ref, w_ref, o3_ref, o5_ref):
    dv = _dv(dinv_ref)
    t = (p_ref[0, :, 0:64] + p_ref[1, :, 0:64] + h_ref[:, 0:64]) * dv \
        + b_ref[...]
    t = jnp.maximum(t, 0.0)
    r = jnp.dot(t * dv, w_ref[...],
                preferred_element_type=jnp.float32, precision=_PREC)
    o3_ref[...] = r[:, 0:D]
    o5_ref[...] = r[:, D:2 * D]


_comb_mm2_call = pl.pallas_call(
    _comb_mm2_body,
    grid=(N // MBLK,),
    in_specs=[
        pl.BlockSpec((NC, MBLK, D), lambda i: (0, i, 0)),
        pl.BlockSpec((MBLK, D), lambda i: (i, 0)),
        _DINV_SPEC,
        pl.BlockSpec((1, 64), lambda i: (0, 0)),
        pl.BlockSpec((64, 2 * D), lambda i: (0, 0)),
    ],
    out_specs=(
        pl.BlockSpec((MBLK, D), lambda i: (i, 0)),
        pl.BlockSpec((MBLK, D), lambda i: (i, 0)),
    ),
    out_shape=(
        jax.ShapeDtypeStruct((N, D), jnp.float32),
        jax.ShapeDtypeStruct((N, D), jnp.float32),
    ),
)


def _s_body(p_ref, h_ref, dinv_ref, b_ref, o_ref):
    dv = _dv(dinv_ref)
    t = (p_ref[0, :, 0:64] + p_ref[1, :, 0:64] + h_ref[:, 0:64]) * dv \
        + b_ref[...]
    o_ref[...] = jnp.maximum(t, 0.0)


_s_call = pl.pallas_call(
    _s_body,
    grid=(N // MBLK,),
    in_specs=[
        pl.BlockSpec((NC, MBLK, D), lambda i: (0, i, 0)),
        pl.BlockSpec((MBLK, D), lambda i: (i, 0)),
        _DINV_SPEC,
        pl.BlockSpec((1, 64), lambda i: (0, 0)),
    ],
    out_specs=pl.BlockSpec((MBLK, 64), lambda i: (i, 0)),
    out_shape=jax.ShapeDtypeStruct((N, 64), jnp.float32),
)


def _xhat_body(p_ref, h_ref, dinv_ref, b_ref, o_ref):
    dv = _dv(dinv_ref)
    o_ref[...] = (p_ref[0] + p_ref[1] + h_ref[...]) * dv + b_ref[...]


_xhat_call = pl.pallas_call(
    _xhat_body,
    grid=(N // MBLK,),
    in_specs=[
        pl.BlockSpec((NC, MBLK, D), lambda i: (0, i, 0)),
        pl.BlockSpec((MBLK, D), lambda i: (i, 0)),
        _DINV_SPEC,
        pl.BlockSpec((1, D), lambda i: (0, 0)),
    ],
    out_specs=pl.BlockSpec((MBLK, D), lambda i: (i, 0)),
    out_shape=jax.ShapeDtypeStruct((N, D), jnp.float32),
)


def _ahat_body(si_ref, sj_ref, o_ref):
    o_ref[...] = lax.dot_general(
        si_ref[...], sj_ref[...], (((1,), (1,)), ((), ())),
        preferred_element_type=jnp.float32, precision=_PREC)


ABLK = 1024

_ahat_call = pl.pallas_call(
    _ahat_body,
    grid=(pl.cdiv(N, ABLK), pl.cdiv(N, ABLK)),
    in_specs=[
        pl.BlockSpec((ABLK, 64), lambda i, j: (i, 0)),
        pl.BlockSpec((ABLK, 64), lambda i, j: (j, 0)),
    ],
    out_specs=pl.BlockSpec((ABLK, ABLK), lambda i, j: (i, j)),
    out_shape=jax.ShapeDtypeStruct((N, N), jnp.float32),
)


def kernel(x, edge_index, W1, b1, W2, b2, W3, b3, W4, b4, W5, b5):
    src = edge_index[0]
    dst = edge_index[1]

    degp = _sc_degree(dst.reshape(E // CHUNK, 1, CHUNK))
    dinv = _dinv_call(degp)

    z64 = jnp.zeros((64, 64), jnp.float32)
    W2p = jnp.concatenate([W2, jnp.zeros((D, 64), jnp.float32)], axis=1)
    Wcat = jnp.concatenate([W3, W5, z64], axis=1)

    h1 = _pre_mm_call(x, dinv, W1)
    p1 = _agg(h1, src, dst)
    h2 = _comb_mm_call(p1, h1, dinv, b1.reshape(1, -1), W2p)
    p2 = _agg(h2, src, dst)

    h3, h5 = _comb_mm2_call(p2, h2, dinv, b2.reshape(1, -1), Wcat)
    p5 = _agg(h5, src, dst)
    s = _s_call(p5, h5, dinv, b5.reshape(1, -1))

    p3 = _agg(h3, src, dst)
    a_hat = _ahat_call(s, s)
    h4 = _comb_mm_call(p3, h3, dinv, b3.reshape(1, -1), W4)
    p4 = _agg(h4, src, dst)
    x_hat = _xhat_call(p4, h4, dinv, b4.reshape(1, -1))
    return (x_hat, a_hat)
